# baseline (speedup 1.0000x reference)
import functools

import jax
import jax.numpy as jnp
from jax import lax
from jax.experimental import pallas as pl
from jax.experimental.pallas import tpu as pltpu
from jax.experimental.pallas import tpu_sc as plsc

N_NODES = 10000
SEQ_LEN = 32
IN_CH = 128
OUT_CH = 128

NUM_CORES = 2
NUM_SUBCORES = 16
NUM_WORKERS = NUM_CORES * NUM_SUBCORES
ROWS_PER_WORKER = N_NODES

CHUNK = 200
N_CHUNKS = ROWS_PER_WORKER // CHUNK
LANES = 16
PAIRS = CHUNK // 2
PAIR_ROWS_PER_WORKER = ROWS_PER_WORKER // 2


def _sc_gather_body(table_hbm, idx_hbm, out_hbm, idx_all,
                    rows_v0, rows_v1, bf_v0, bf_v1,
                    gsem0, gsem1, wsem0, wsem1):
    rows_v = (rows_v0, rows_v1)
    bf_v = (bf_v0, bf_v1)
    gsem = (gsem0, gsem1)
    wsem = (wsem0, wsem1)
    wid = lax.axis_index("s") * NUM_CORES + lax.axis_index("c")
    base = wid * ROWS_PER_WORKER
    pbase = wid * PAIR_ROWS_PER_WORKER

    pltpu.sync_copy(idx_hbm.at[pl.ds(base, ROWS_PER_WORKER)], idx_all)

    def start_gather(c):
        fb = c % 2
        pltpu.make_async_copy(
            table_hbm.at[idx_all.at[pl.ds(c * CHUNK, CHUNK)]],
            rows_v[fb], gsem[fb]).start()

    def convert(c):
        src = rows_v[c % 2]
        dst = bf_v[(c // 2) % 2]
        half = c % 2

        rnd = jnp.int32(0x8000)
        mask_hi = jnp.int32(-65536)

        def pair_body(jj, carry):
            j0 = 2 * jj
            for k in range(IN_CH // LANES):
                a = src[j0, pl.ds(LANES * k, LANES)]
                b2 = src[j0 + 1, pl.ds(LANES * k, LANES)]
                au = lax.bitcast_convert_type(a, jnp.int32)
                bu = lax.bitcast_convert_type(b2, jnp.int32)
                lo = lax.shift_right_logical(au + rnd, jnp.int32(16))
                hi = (bu + rnd) & mask_hi
                dst[half * PAIRS + jj, pl.ds(LANES * k, LANES)] = lo | hi
            return carry

        lax.fori_loop(0, PAIRS, pair_body, 0)

    start_gather(0)
    start_gather(1)
    for c in range(N_CHUNKS):
        fb = c % 2
        B = (c // 2) % 2
        pltpu.make_async_copy(
            table_hbm.at[idx_all.at[pl.ds(c * CHUNK, CHUNK)]],
            rows_v[fb], gsem[fb]).wait()
        if c % 2 == 0 and c >= 4:
            pltpu.make_async_copy(
                bf_v[B],
                out_hbm.at[pl.ds(pbase + (c // 2 - 2) * CHUNK, CHUNK)],
                wsem[B]).wait()
        convert(c)
        if c + 2 < N_CHUNKS:
            start_gather(c + 2)
        if c % 2 == 1:
            pltpu.make_async_copy(
                bf_v[B], out_hbm.at[pl.ds(pbase + (c // 2) * CHUNK, CHUNK)],
                wsem[B]).start()
    for p in (N_CHUNKS // 2 - 2, N_CHUNKS // 2 - 1):
        B = p % 2
        pltpu.make_async_copy(
            bf_v[B], out_hbm.at[pl.ds(pbase + p * CHUNK, CHUNK)],
            wsem[B]).wait()


def _sc_gather(x, idx_flat):
    mesh = plsc.VectorSubcoreMesh(core_axis_name="c", subcore_axis_name="s")
    kfn = pl.kernel(
        _sc_gather_body,
        mesh=mesh,
        out_type=jax.ShapeDtypeStruct((SEQ_LEN * N_NODES // 2, IN_CH),
                                      jnp.int32),
        scratch_types=(
            [pltpu.VMEM((ROWS_PER_WORKER,), jnp.int32)]
            + [pltpu.VMEM((CHUNK, IN_CH), jnp.float32)] * 2
            + [pltpu.VMEM((CHUNK, IN_CH), jnp.int32)] * 2
            + [pltpu.SemaphoreType.DMA] * 4
        ),
    )
    return kfn(x, idx_flat)


def _mm_body(g_ref, wt_ref, b_ref, o_ref):
    acc = jnp.broadcast_to(b_ref[...], o_ref.shape)
    for s in range(SEQ_LEN):
        gs = pltpu.bitcast(g_ref[s], jnp.bfloat16)
        acc = acc + lax.dot_general(
            gs, wt_ref[s],
            (((1,), (0,)), ((), ())),
            preferred_element_type=jnp.float32,
        )
    o_ref[...] = acc


def _tc_matmul(g3, Wt, b):
    m_block = 2000
    grid = (N_NODES // m_block,)
    return pl.pallas_call(
        _mm_body,
        grid=grid,
        in_specs=[
            pl.BlockSpec((SEQ_LEN, m_block // 2, IN_CH), lambda i: (0, i, 0)),
            pl.BlockSpec((SEQ_LEN, IN_CH, OUT_CH), lambda i: (0, 0, 0)),
            pl.BlockSpec((1, OUT_CH), lambda i: (0, 0)),
        ],
        out_specs=pl.BlockSpec((m_block, OUT_CH), lambda i: (i, 0)),
        out_shape=jax.ShapeDtypeStruct((N_NODES, OUT_CH), jnp.float32),
    )(g3, Wt, b)


@jax.jit
def kernel(x, indices, W, b):
    idx_flat = indices.astype(jnp.int32).T.reshape(-1)
    Wt = W.reshape(OUT_CH, SEQ_LEN, IN_CH).transpose(1, 2, 0)
    Wt = Wt.astype(jnp.bfloat16)
    g = _sc_gather(x, idx_flat)
    g3 = g.reshape(SEQ_LEN, PAIR_ROWS_PER_WORKER, IN_CH)
    return _tc_matmul(g3, Wt, b.reshape(1, OUT_CH))


# speedup vs baseline: 1.5826x; 1.5826x over previous
import functools

import jax
import jax.numpy as jnp
from jax import lax
from jax.experimental import pallas as pl
from jax.experimental.pallas import tpu as pltpu
from jax.experimental.pallas import tpu_sc as plsc

N_NODES = 10000
SEQ_LEN = 32
IN_CH = 128
OUT_CH = 128

NUM_CORES = 2
NUM_SUBCORES = 16
NUM_WORKERS = NUM_CORES * NUM_SUBCORES
ROWS_PER_WORKER = N_NODES

CHUNK = 400
N_CHUNKS = ROWS_PER_WORKER // CHUNK
NBUF = 2


def _sc_gather_body(table_hbm, idx_hbm, out_hbm, idx_all, *bufs):
    rows_v = bufs[:NBUF]
    gsem = bufs[NBUF:2 * NBUF]
    wsem = bufs[2 * NBUF:3 * NBUF]
    wid = lax.axis_index("s") * NUM_CORES + lax.axis_index("c")
    base = wid * ROWS_PER_WORKER

    pltpu.sync_copy(idx_hbm.at[pl.ds(base, ROWS_PER_WORKER)], idx_all)

    def start_gather(c):
        b = c % NBUF
        pltpu.make_async_copy(
            table_hbm.at[idx_all.at[pl.ds(c * CHUNK, CHUNK)]],
            rows_v[b], gsem[b]).start()

    for c in range(NBUF):
        start_gather(c)
    for c in range(N_CHUNKS):
        b = c % NBUF
        pltpu.make_async_copy(
            table_hbm.at[idx_all.at[pl.ds(c * CHUNK, CHUNK)]],
            rows_v[b], gsem[b]).wait()
        wb = pltpu.make_async_copy(
            rows_v[b], out_hbm.at[pl.ds(base + c * CHUNK, CHUNK)], wsem[b])
        wb.start()
        if c + NBUF < N_CHUNKS:
            wb.wait()
            start_gather(c + NBUF)
        else:
            wb.wait()


def _sc_gather(x, idx_flat):
    mesh = plsc.VectorSubcoreMesh(core_axis_name="c", subcore_axis_name="s")
    kfn = pl.kernel(
        _sc_gather_body,
        mesh=mesh,
        out_type=jax.ShapeDtypeStruct((SEQ_LEN * N_NODES, IN_CH), jnp.float32),
        scratch_types=(
            [pltpu.VMEM((ROWS_PER_WORKER,), jnp.int32)]
            + [pltpu.VMEM((CHUNK, IN_CH), jnp.float32)] * NBUF
            + [pltpu.SemaphoreType.DMA] * (2 * NBUF)
        ),
    )
    return kfn(x, idx_flat)


def _mm_body(g_ref, wt_ref, b_ref, o_ref):
    acc = jnp.broadcast_to(b_ref[...], o_ref.shape)
    for s in range(SEQ_LEN):
        acc = acc + lax.dot_general(
            g_ref[s].astype(jnp.bfloat16), wt_ref[s].astype(jnp.bfloat16),
            (((1,), (0,)), ((), ())),
            preferred_element_type=jnp.float32,
        )
    o_ref[...] = acc


def _tc_matmul(gout, Wt, b):
    m_block = 1000
    grid = (N_NODES // m_block,)
    return pl.pallas_call(
        _mm_body,
        grid=grid,
        in_specs=[
            pl.BlockSpec((SEQ_LEN, m_block, IN_CH), lambda i: (0, i, 0)),
            pl.BlockSpec((SEQ_LEN, IN_CH, OUT_CH), lambda i: (0, 0, 0)),
            pl.BlockSpec((1, OUT_CH), lambda i: (0, 0)),
        ],
        out_specs=pl.BlockSpec((m_block, OUT_CH), lambda i: (i, 0)),
        out_shape=jax.ShapeDtypeStruct((N_NODES, OUT_CH), jnp.float32),
    )(gout, Wt, b)


@jax.jit
def kernel(x, indices, W, b):
    idx_flat = indices.astype(jnp.int32).T.reshape(-1)
    Wt = W.reshape(OUT_CH, SEQ_LEN, IN_CH).transpose(1, 2, 0)
    g = _sc_gather(x, idx_flat)
    gout = g.reshape(SEQ_LEN, N_NODES, IN_CH)
    return _tc_matmul(gout, Wt, b.reshape(1, OUT_CH))
